# Initial kernel scaffold; baseline (speedup 1.0000x reference)
import jax
import jax.numpy as jnp
from jax.experimental import pallas as pl


def kernel(x, solutions, best_solutions):
    raise NotImplementedError("write your pallas kernel here")


# speedup vs baseline: 11.5186x; 11.5186x over previous
import functools

import numpy as np
import jax
import jax.numpy as jnp
from jax import lax
from jax.experimental import pallas as pl
from jax.experimental.pallas import tpu as pltpu
from jax.experimental.pallas import tpu_sc as plsc

EMB_DIM = 128
SEQ = 200
HALF = 100
NUM_WORKERS = 32


def _position_encoding_table(n_position, emb_dim):
    pos = np.arange(1, n_position + 1, dtype=np.float64)[:, None]
    j = np.arange(emb_dim, dtype=np.float64)[None, :]
    pe = pos / np.power(10000.0, 2.0 * (np.floor(j / 2.0)) / emb_dim)
    pe[1:, 0::2] = np.sin(pe[1:, 0::2])
    pe[1:, 1::2] = np.cos(pe[1:, 1::2])
    return pe.astype(np.float32)


_ENC = _position_encoding_table(SEQ, EMB_DIM)


@functools.lru_cache(maxsize=None)
def _make_scatter_kernel(B):
    rows_per = B // NUM_WORKERS
    mesh = plsc.VectorSubcoreMesh(core_axis_name="c", subcore_axis_name="s")

    @functools.partial(
        pl.kernel,
        mesh=mesh,
        out_type=(
            jax.ShapeDtypeStruct((B, SEQ, EMB_DIM), jnp.float32),
            jax.ShapeDtypeStruct((B, SEQ, EMB_DIM), jnp.float32),
        ),
        scratch_types=[
            pltpu.VMEM((SEQ, EMB_DIM), jnp.float32),
            pltpu.VMEM((2, HALF), jnp.int32),
            pltpu.VMEM((2, HALF), jnp.int32),
            pltpu.SemaphoreType.DMA,
        ],
    )
    def scatter_kernel(enc_hbm, sol_hbm, best_hbm, out0, out1,
                       enc_v, idx0, idx1, sem):
        wid = lax.axis_index("s") * 2 + lax.axis_index("c")
        pltpu.sync_copy(enc_hbm, enc_v)
        base = wid * rows_per

        def body(i, carry):
            b = base + i
            pltpu.sync_copy(sol_hbm.at[b], idx0)
            pltpu.sync_copy(best_hbm.at[b], idx1)
            copies = []
            for j in range(2):
                src = enc_v.at[pl.ds(j * HALF, HALF)]
                copies.append(
                    pltpu.async_copy(src, out0.at[b].at[idx0.at[j]], sem))
                copies.append(
                    pltpu.async_copy(src, out1.at[b].at[idx1.at[j]], sem))
            for cp in copies:
                cp.wait()
            return carry

        lax.fori_loop(0, rows_per, body, 0)

    return scatter_kernel


def kernel(x, solutions, best_solutions):
    B, S = solutions.shape
    enc = jnp.asarray(_ENC)
    sol = solutions.astype(jnp.int32).reshape(B, 2, HALF)
    best = best_solutions.astype(jnp.int32).reshape(B, 2, HALF)
    return _make_scatter_kernel(B)(enc, sol, best)


# speedup vs baseline: 15.3822x; 1.3354x over previous
import functools

import numpy as np
import jax
import jax.numpy as jnp
from jax import lax
from jax.experimental import pallas as pl
from jax.experimental.pallas import tpu as pltpu
from jax.experimental.pallas import tpu_sc as plsc

EMB_DIM = 128
SEQ = 200
HALF = 100
NUM_WORKERS = 32


def _position_encoding_table(n_position, emb_dim):
    pos = np.arange(1, n_position + 1, dtype=np.float64)[:, None]
    j = np.arange(emb_dim, dtype=np.float64)[None, :]
    pe = pos / np.power(10000.0, 2.0 * (np.floor(j / 2.0)) / emb_dim)
    pe[1:, 0::2] = np.sin(pe[1:, 0::2])
    pe[1:, 1::2] = np.cos(pe[1:, 1::2])
    return pe.astype(np.float32)


_ENC = _position_encoding_table(SEQ, EMB_DIM)


@functools.lru_cache(maxsize=None)
def _make_scatter_kernel(B):
    rows_per = B // NUM_WORKERS
    mesh = plsc.VectorSubcoreMesh(core_axis_name="c", subcore_axis_name="s")

    @functools.partial(
        pl.kernel,
        mesh=mesh,
        out_type=(
            jax.ShapeDtypeStruct((B, SEQ, EMB_DIM), jnp.float32),
            jax.ShapeDtypeStruct((B, SEQ, EMB_DIM), jnp.float32),
        ),
        scratch_types=[
            pltpu.VMEM((SEQ, EMB_DIM), jnp.float32),
            pltpu.VMEM((rows_per, 2, HALF), jnp.int32),
            pltpu.VMEM((rows_per, 2, HALF), jnp.int32),
            pltpu.SemaphoreType.DMA,
        ],
    )
    def scatter_kernel(enc_hbm, sol_hbm, best_hbm, out0, out1,
                       enc_v, idx0, idx1, sem):
        wid = lax.axis_index("s") * 2 + lax.axis_index("c")
        base = wid * rows_per
        pltpu.sync_copy(enc_hbm, enc_v)
        pltpu.sync_copy(sol_hbm.at[pl.ds(base, rows_per)], idx0)
        pltpu.sync_copy(best_hbm.at[pl.ds(base, rows_per)], idx1)

        def body(i, carry):
            b = base + i
            for j in range(2):
                src = enc_v.at[pl.ds(j * HALF, HALF)]
                pltpu.async_copy(src, out0.at[b].at[idx0.at[i, j]], sem)
                pltpu.async_copy(src, out1.at[b].at[idx1.at[i, j]], sem)
            return carry

        lax.fori_loop(0, rows_per, body, 0)

        def drain(i, carry):
            pltpu.make_async_copy(out0.at[0], enc_v, sem).wait()
            return carry

        lax.fori_loop(0, 2 * rows_per, drain, 0)

    return scatter_kernel


def kernel(x, solutions, best_solutions):
    B, S = solutions.shape
    enc = jnp.asarray(_ENC)
    sol = solutions.astype(jnp.int32).reshape(B, 2, HALF)
    best = best_solutions.astype(jnp.int32).reshape(B, 2, HALF)
    return _make_scatter_kernel(B)(enc, sol, best)
